# baseline (device time: 88136 ns/iter reference)
import jax
import jax.numpy as jnp
from jax import lax
from jax.experimental import pallas as pl
from jax.experimental.pallas import tpu as pltpu

N_DEV = 16
SQ = 1024
SKV = 1024
D_MODEL = 1024
HALF_D = D_MODEL // 2
HEADS_PER_SHARD = 8
DH = 128
WINDOW = 128
SCALE = 0.08838834764831843

MASKS_A = (1, 2, 4, 8)
MASKS_B = (4, 8, 2, 1)
CONTRIB_A = {1: 512, 2: 256, 4: 128, 8: 64}
CONTRIB_B = {4: 512, 8: 256, 2: 128, 1: 64}


def _body(x_ref, wq_ref, k_ref, v_ref, wo_ref, out_ref,
          q_ref, ctx_ref, kt_ref, vt_ref, acc_ref,
          accb_ref, rbufa_ref, rbufb_ref, gbufa_ref, gbufb_ref,
          rsa_send, rsa_recv, rsb_send, rsb_recv,
          dba_send, dba_recv, dbb_send, dbb_recv):
    my = lax.axis_index("i")

    q = lax.dot_general(
        x_ref[...], wq_ref[...], (((1,), (0,)), ((), ())),
        preferred_element_type=jnp.float32,
    )
    q_ref[...] = q.astype(jnp.bfloat16)

    for h in range(HEADS_PER_SHARD):
        kt_ref[h, :, :] = k_ref[:, h, :]
        vt_ref[h, :, :] = v_ref[:, h, :]

    RBLK = 256
    for h in range(HEADS_PER_SHARD):
        for r in range(SQ // RBLK):
            r0 = r * RBLK
            c0 = max(0, r0 - WINDOW)
            c1 = min(SKV, r0 + RBLK + WINDOW)
            w = c1 - c0
            qblk = q_ref[r0:r0 + RBLK, h * DH:(h + 1) * DH]
            scores = lax.dot_general(
                qblk, kt_ref[h, c0:c1, :], (((1,), (1,)), ((), ())),
                preferred_element_type=jnp.float32,
            ) * SCALE
            rows = lax.broadcasted_iota(jnp.int32, (RBLK, w), 0) + r0
            cols = lax.broadcasted_iota(jnp.int32, (RBLK, w), 1) + c0
            scores = jnp.where(jnp.abs(rows - cols) <= WINDOW, scores, -1e9)
            m = jnp.max(scores, axis=1, keepdims=True)
            e = jnp.exp(scores - m)
            s = jnp.sum(e, axis=1, keepdims=True)
            wgt = (e / s).astype(jnp.bfloat16)
            ctx = lax.dot_general(
                wgt, vt_ref[h, c0:c1, :], (((1,), (0,)), ((), ())),
                preferred_element_type=jnp.float32,
            )
            ctx_ref[r0:r0 + RBLK, h * DH:(h + 1) * DH] = ctx.astype(
                jnp.bfloat16
            )

    accv = lax.dot_general(
        ctx_ref[...], wo_ref[...], (((1,), (0,)), ((), ())),
        preferred_element_type=jnp.float32,
    )
    acc_ref[...] = accv
    accb_ref[...] = accv.astype(jnp.bfloat16)

    bsem = pltpu.get_barrier_semaphore()
    for mk in MASKS_A:
        pl.semaphore_signal(bsem, inc=1, device_id=(my ^ mk,),
                            device_id_type=pl.DeviceIdType.MESH)
    pl.semaphore_wait(bsem, 4)

    def _keep_give(start, half, mk):
        upper = (my & mk) != 0
        keep = pl.multiple_of(
            jnp.where(upper, start + half, start).astype(jnp.int32), 64
        )
        give = pl.multiple_of(
            jnp.where(upper, start, start + half).astype(jnp.int32), 64
        )
        return keep, give

    start_a = jnp.int32(0)
    start_b = jnp.int32(0)
    length = SQ
    for k in range(4):
        half = length // 2
        mka, mkb = MASKS_A[k], MASKS_B[k]
        keep_a, give_a = _keep_give(start_a, half, mka)
        keep_b, give_b = _keep_give(start_b, half, mkb)
        step_a = pltpu.make_async_remote_copy(
            src_ref=accb_ref.at[pl.ds(give_a, half), 0:HALF_D],
            dst_ref=rbufa_ref.at[k, 0:half, :],
            send_sem=rsa_send.at[k], recv_sem=rsa_recv.at[k],
            device_id=(my ^ mka,), device_id_type=pl.DeviceIdType.MESH,
        )
        step_b = pltpu.make_async_remote_copy(
            src_ref=accb_ref.at[pl.ds(give_b, half), HALF_D:D_MODEL],
            dst_ref=rbufb_ref.at[k, 0:half, :],
            send_sem=rsb_send.at[k], recv_sem=rsb_recv.at[k],
            device_id=(my ^ mkb,), device_id_type=pl.DeviceIdType.MESH,
        )
        step_a.start()
        step_b.start()
        step_a.wait_send()
        step_a.wait_recv()
        new_a = (
            acc_ref[pl.ds(keep_a, half), 0:HALF_D]
            + rbufa_ref[k, 0:half, :].astype(jnp.float32)
        )
        acc_ref[pl.ds(keep_a, half), 0:HALF_D] = new_a
        accb_ref[pl.ds(keep_a, half), 0:HALF_D] = new_a.astype(jnp.bfloat16)
        step_b.wait_send()
        step_b.wait_recv()
        new_b = (
            acc_ref[pl.ds(keep_b, half), HALF_D:D_MODEL]
            + rbufb_ref[k, 0:half, :].astype(jnp.float32)
        )
        acc_ref[pl.ds(keep_b, half), HALF_D:D_MODEL] = new_b
        accb_ref[pl.ds(keep_b, half), HALF_D:D_MODEL] = new_b.astype(
            jnp.bfloat16
        )
        start_a, start_b = keep_a, keep_b
        length = half

    gbufa_ref[pl.ds(start_a, 64), :] = accb_ref[pl.ds(start_a, 64), 0:HALF_D]
    gbufb_ref[pl.ds(start_b, 64), :] = accb_ref[
        pl.ds(start_b, 64), HALF_D:D_MODEL]
    cur_a, cur_b = start_a, start_b
    cur_len = 64
    for k in range(4):
        mka = MASKS_A[3 - k]
        mkb = MASKS_B[3 - k]
        pstart_a = pl.multiple_of(
            jnp.where((my & mka) != 0, cur_a - CONTRIB_A[mka],
                      cur_a + CONTRIB_A[mka]).astype(jnp.int32), 64)
        pstart_b = pl.multiple_of(
            jnp.where((my & mkb) != 0, cur_b - CONTRIB_B[mkb],
                      cur_b + CONTRIB_B[mkb]).astype(jnp.int32), 64)
        send_a = pltpu.make_async_remote_copy(
            src_ref=gbufa_ref.at[pl.ds(cur_a, cur_len), :],
            dst_ref=gbufa_ref.at[pl.ds(cur_a, cur_len), :],
            send_sem=dba_send.at[k], recv_sem=dba_recv.at[k],
            device_id=(my ^ mka,), device_id_type=pl.DeviceIdType.MESH,
        )
        send_b = pltpu.make_async_remote_copy(
            src_ref=gbufb_ref.at[pl.ds(cur_b, cur_len), :],
            dst_ref=gbufb_ref.at[pl.ds(cur_b, cur_len), :],
            send_sem=dbb_send.at[k], recv_sem=dbb_recv.at[k],
            device_id=(my ^ mkb,), device_id_type=pl.DeviceIdType.MESH,
        )
        send_a.start()
        send_b.start()
        send_a.wait_send()
        recv_a = pltpu.make_async_remote_copy(
            src_ref=gbufa_ref.at[pl.ds(pstart_a, cur_len), :],
            dst_ref=gbufa_ref.at[pl.ds(pstart_a, cur_len), :],
            send_sem=dba_send.at[k], recv_sem=dba_recv.at[k],
            device_id=(my ^ mka,), device_id_type=pl.DeviceIdType.MESH,
        )
        recv_a.wait_recv()
        send_b.wait_send()
        recv_b = pltpu.make_async_remote_copy(
            src_ref=gbufb_ref.at[pl.ds(pstart_b, cur_len), :],
            dst_ref=gbufb_ref.at[pl.ds(pstart_b, cur_len), :],
            send_sem=dbb_send.at[k], recv_sem=dbb_recv.at[k],
            device_id=(my ^ mkb,), device_id_type=pl.DeviceIdType.MESH,
        )
        recv_b.wait_recv()
        cur_a = pl.multiple_of(jnp.minimum(cur_a, pstart_a), 64)
        cur_b = pl.multiple_of(jnp.minimum(cur_b, pstart_b), 64)
        cur_len *= 2

    out_ref[:, 0:HALF_D] = gbufa_ref[...].astype(jnp.float32)
    out_ref[:, HALF_D:D_MODEL] = gbufb_ref[...].astype(jnp.float32)


def kernel(x, Wq, K_ext, V_ext, Wo):
    pos = lax.axis_index("i")
    xb = x[0].astype(jnp.bfloat16)
    wq = Wq.astype(jnp.bfloat16)
    wo = Wo.astype(jnp.bfloat16)
    kh = lax.dynamic_slice(
        K_ext, (0, 0, pos * HEADS_PER_SHARD, 0), (1, SKV, HEADS_PER_SHARD, DH)
    )[0]
    vh = lax.dynamic_slice(
        V_ext, (0, 0, pos * HEADS_PER_SHARD, 0), (1, SKV, HEADS_PER_SHARD, DH)
    )[0]
    kh = kh.astype(jnp.bfloat16)
    vh = vh.astype(jnp.bfloat16)

    out = pl.pallas_call(
        _body,
        out_shape=jax.ShapeDtypeStruct((SQ, D_MODEL), jnp.float32),
        in_specs=[pl.BlockSpec(memory_space=pltpu.VMEM)] * 5,
        out_specs=pl.BlockSpec(memory_space=pltpu.VMEM),
        scratch_shapes=[
            pltpu.VMEM((SQ, D_MODEL), jnp.bfloat16),
            pltpu.VMEM((SQ, D_MODEL), jnp.bfloat16),
            pltpu.VMEM((HEADS_PER_SHARD, SKV, DH), jnp.bfloat16),
            pltpu.VMEM((HEADS_PER_SHARD, SKV, DH), jnp.bfloat16),
            pltpu.VMEM((SQ, D_MODEL), jnp.float32),
            pltpu.VMEM((SQ, D_MODEL), jnp.bfloat16),
            pltpu.VMEM((4, SQ // 2, HALF_D), jnp.bfloat16),
            pltpu.VMEM((4, SQ // 2, HALF_D), jnp.bfloat16),
            pltpu.VMEM((SQ, HALF_D), jnp.bfloat16),
            pltpu.VMEM((SQ, HALF_D), jnp.bfloat16),
            pltpu.SemaphoreType.DMA((4,)),
            pltpu.SemaphoreType.DMA((4,)),
            pltpu.SemaphoreType.DMA((4,)),
            pltpu.SemaphoreType.DMA((4,)),
            pltpu.SemaphoreType.DMA((4,)),
            pltpu.SemaphoreType.DMA((4,)),
            pltpu.SemaphoreType.DMA((4,)),
            pltpu.SemaphoreType.DMA((4,)),
        ],
        compiler_params=pltpu.CompilerParams(collective_id=0),
    )(xb, wq, kh, vh, wo)
    return out.reshape(1, SQ, D_MODEL)


# device time: 80872 ns/iter; 1.0898x vs baseline; 1.0898x over previous
import jax
import jax.numpy as jnp
from jax import lax
from jax.experimental import pallas as pl
from jax.experimental.pallas import tpu as pltpu

N_DEV = 16
SQ = 1024
SKV = 1024
D_MODEL = 1024
HALF_D = D_MODEL // 2
HEADS_PER_SHARD = 8
DH = 128
WINDOW = 128
SCALE = 0.08838834764831843

MASKS_A = (1, 2, 4, 8)
MASKS_B = (4, 8, 2, 1)
CONTRIB_A = {1: 512, 2: 256, 4: 128, 8: 64}
CONTRIB_B = {4: 512, 8: 256, 2: 128, 1: 64}
HALVES = (512, 256, 128, 64)


def _body(x_ref, wq_ref, k_ref, v_ref, wo_ref, out_ref,
          q_ref, ctx_ref, acc_ref,
          accb_ref, rbufa_ref, rbufb_ref, gbufa_ref, gbufb_ref,
          rsa_send, rsa_recv, rsb_send, rsb_recv,
          dba_send, dba_recv, dbb_send, dbb_recv):
    my = lax.axis_index("i")

    q = lax.dot_general(
        x_ref[...], wq_ref[...], (((1,), (0,)), ((), ())),
        preferred_element_type=jnp.float32,
    )
    q_ref[...] = q.astype(jnp.bfloat16)

    RBLK = 256
    for h in range(HEADS_PER_SHARD):
        for r in range(SQ // RBLK):
            r0 = r * RBLK
            c0 = max(0, r0 - WINDOW)
            c1 = min(SKV, r0 + RBLK + WINDOW)
            w = c1 - c0
            qblk = q_ref[r0:r0 + RBLK, h * DH:(h + 1) * DH]
            scores = lax.dot_general(
                qblk, k_ref[h, c0:c1, :], (((1,), (1,)), ((), ())),
                preferred_element_type=jnp.float32,
            ) * SCALE
            rows = lax.broadcasted_iota(jnp.int32, (RBLK, w), 0) + r0
            cols = lax.broadcasted_iota(jnp.int32, (RBLK, w), 1) + c0
            scores = jnp.where(jnp.abs(rows - cols) <= WINDOW, scores, -1e9)
            m = jnp.max(scores, axis=1, keepdims=True)
            e = jnp.exp(scores - m)
            s = jnp.sum(e, axis=1, keepdims=True)
            wgt = (e / s).astype(jnp.bfloat16)
            ctx = lax.dot_general(
                wgt, v_ref[h, c0:c1, :], (((1,), (0,)), ((), ())),
                preferred_element_type=jnp.float32,
            )
            ctx_ref[r0:r0 + RBLK, h * DH:(h + 1) * DH] = ctx.astype(
                jnp.bfloat16
            )

    accv = lax.dot_general(
        ctx_ref[...], wo_ref[...], (((1,), (0,)), ((), ())),
        preferred_element_type=jnp.float32,
    )
    acc_ref[...] = accv
    accb_ref[...] = accv.astype(jnp.bfloat16)

    bsem = pltpu.get_barrier_semaphore()
    for mk in MASKS_A:
        pl.semaphore_signal(bsem, inc=1, device_id=(my ^ mk,),
                            device_id_type=pl.DeviceIdType.MESH)
    pl.semaphore_wait(bsem, 4)

    def _keep_give(start, half, mk):
        upper = (my & mk) != 0
        keep = pl.multiple_of(
            jnp.where(upper, start + half, start).astype(jnp.int32), 64
        )
        give = pl.multiple_of(
            jnp.where(upper, start, start + half).astype(jnp.int32), 64
        )
        return keep, give

    def _rs_issue(k, give, mk, cols, rbuf, send_sems, recv_sems):
        half = HALVES[k]
        step = pltpu.make_async_remote_copy(
            src_ref=accb_ref.at[pl.ds(give, half), cols],
            dst_ref=rbuf.at[k, 0:half, :],
            send_sem=send_sems.at[k], recv_sem=recv_sems.at[k],
            device_id=(my ^ mk,), device_id_type=pl.DeviceIdType.MESH,
        )
        step.start()
        return step

    COLS_A = slice(0, HALF_D)
    COLS_B = slice(HALF_D, D_MODEL)

    keeps_a = [None] * 4
    keeps_b = [None] * 4
    gives_a = [None] * 4
    gives_b = [None] * 4
    sa, sb = jnp.int32(0), jnp.int32(0)
    for k in range(4):
        keeps_a[k], gives_a[k] = _keep_give(sa, HALVES[k], MASKS_A[k])
        keeps_b[k], gives_b[k] = _keep_give(sb, HALVES[k], MASKS_B[k])
        sa, sb = keeps_a[k], keeps_b[k]
    start_a, start_b = sa, sb

    def _rs_add(k, keep, cols, rbuf):
        half = HALVES[k]
        new = (
            acc_ref[pl.ds(keep, half), cols]
            + rbuf[k, 0:half, :].astype(jnp.float32)
        )
        acc_ref[pl.ds(keep, half), cols] = new
        accb_ref[pl.ds(keep, half), cols] = new.astype(jnp.bfloat16)

    steps_a = [None] * 4
    steps_b = [None] * 4
    steps_a[0] = _rs_issue(0, gives_a[0], MASKS_A[0], COLS_A,
                           rbufa_ref, rsa_send, rsa_recv)
    steps_b[0] = _rs_issue(0, gives_b[0], MASKS_B[0], COLS_B,
                           rbufb_ref, rsb_send, rsb_recv)
    for k in range(4):
        steps_a[k].wait_recv()
        _rs_add(k, keeps_a[k], COLS_A, rbufa_ref)
        if k + 1 < 4:
            steps_a[k + 1] = _rs_issue(k + 1, gives_a[k + 1], MASKS_A[k + 1],
                                       COLS_A, rbufa_ref, rsa_send, rsa_recv)
        steps_b[k].wait_recv()
        _rs_add(k, keeps_b[k], COLS_B, rbufb_ref)
        if k + 1 < 4:
            steps_b[k + 1] = _rs_issue(k + 1, gives_b[k + 1], MASKS_B[k + 1],
                                       COLS_B, rbufb_ref, rsb_send, rsb_recv)
    for k in range(4):
        steps_a[k].wait_send()
        steps_b[k].wait_send()

    gbufa_ref[pl.ds(start_a, 64), :] = accb_ref[pl.ds(start_a, 64), COLS_A]
    gbufb_ref[pl.ds(start_b, 64), :] = accb_ref[pl.ds(start_b, 64), COLS_B]

    def _db_desc(k, start, ln, mk, gbuf, send_sems, recv_sems):
        return pltpu.make_async_remote_copy(
            src_ref=gbuf.at[pl.ds(start, ln), :],
            dst_ref=gbuf.at[pl.ds(start, ln), :],
            send_sem=send_sems.at[k], recv_sem=recv_sems.at[k],
            device_id=(my ^ mk,), device_id_type=pl.DeviceIdType.MESH,
        )

    cur_a, cur_b = start_a, start_b
    curs_a = [None] * 4
    curs_b = [None] * 4
    parts_a = [None] * 4
    parts_b = [None] * 4
    lens = (64, 128, 256, 512)
    for k in range(4):
        mka, mkb = MASKS_A[3 - k], MASKS_B[3 - k]
        curs_a[k], curs_b[k] = cur_a, cur_b
        parts_a[k] = pl.multiple_of(
            jnp.where((my & mka) != 0, cur_a - CONTRIB_A[mka],
                      cur_a + CONTRIB_A[mka]).astype(jnp.int32), 64)
        parts_b[k] = pl.multiple_of(
            jnp.where((my & mkb) != 0, cur_b - CONTRIB_B[mkb],
                      cur_b + CONTRIB_B[mkb]).astype(jnp.int32), 64)
        cur_a = pl.multiple_of(jnp.minimum(cur_a, parts_a[k]), 64)
        cur_b = pl.multiple_of(jnp.minimum(cur_b, parts_b[k]), 64)

    sends_a = [None] * 4
    sends_b = [None] * 4
    sends_a[0] = _db_desc(0, curs_a[0], lens[0], MASKS_A[3],
                          gbufa_ref, dba_send, dba_recv)
    sends_b[0] = _db_desc(0, curs_b[0], lens[0], MASKS_B[3],
                          gbufb_ref, dbb_send, dbb_recv)
    sends_a[0].start()
    sends_b[0].start()
    for k in range(4):
        mka, mkb = MASKS_A[3 - k], MASKS_B[3 - k]
        recv_a = _db_desc(k, parts_a[k], lens[k], mka,
                          gbufa_ref, dba_send, dba_recv)
        recv_a.wait_recv()
        if k + 1 < 4:
            sends_a[k + 1] = _db_desc(k + 1, curs_a[k + 1], lens[k + 1],
                                      MASKS_A[3 - k - 1],
                                      gbufa_ref, dba_send, dba_recv)
            sends_a[k + 1].start()
        recv_b = _db_desc(k, parts_b[k], lens[k], mkb,
                          gbufb_ref, dbb_send, dbb_recv)
        recv_b.wait_recv()
        if k + 1 < 4:
            sends_b[k + 1] = _db_desc(k + 1, curs_b[k + 1], lens[k + 1],
                                      MASKS_B[3 - k - 1],
                                      gbufb_ref, dbb_send, dbb_recv)
            sends_b[k + 1].start()
    for k in range(4):
        sends_a[k].wait_send()
        sends_b[k].wait_send()

    out_ref[:, COLS_A] = gbufa_ref[...].astype(jnp.float32)
    out_ref[:, COLS_B] = gbufb_ref[...].astype(jnp.float32)


def kernel(x, Wq, K_ext, V_ext, Wo):
    pos = lax.axis_index("i")
    xb = x[0].astype(jnp.bfloat16)
    wq = Wq.astype(jnp.bfloat16)
    wo = Wo.astype(jnp.bfloat16)
    kh = lax.dynamic_slice(
        K_ext, (0, 0, pos * HEADS_PER_SHARD, 0), (1, SKV, HEADS_PER_SHARD, DH)
    )[0]
    vh = lax.dynamic_slice(
        V_ext, (0, 0, pos * HEADS_PER_SHARD, 0), (1, SKV, HEADS_PER_SHARD, DH)
    )[0]
    kh = jnp.transpose(kh, (1, 0, 2)).astype(jnp.bfloat16)
    vh = jnp.transpose(vh, (1, 0, 2)).astype(jnp.bfloat16)

    out = pl.pallas_call(
        _body,
        out_shape=jax.ShapeDtypeStruct((SQ, D_MODEL), jnp.float32),
        in_specs=[pl.BlockSpec(memory_space=pltpu.VMEM)] * 5,
        out_specs=pl.BlockSpec(memory_space=pltpu.VMEM),
        scratch_shapes=[
            pltpu.VMEM((SQ, D_MODEL), jnp.bfloat16),
            pltpu.VMEM((SQ, D_MODEL), jnp.bfloat16),
            pltpu.VMEM((SQ, D_MODEL), jnp.float32),
            pltpu.VMEM((SQ, D_MODEL), jnp.bfloat16),
            pltpu.VMEM((4, SQ // 2, HALF_D), jnp.bfloat16),
            pltpu.VMEM((4, SQ // 2, HALF_D), jnp.bfloat16),
            pltpu.VMEM((SQ, HALF_D), jnp.bfloat16),
            pltpu.VMEM((SQ, HALF_D), jnp.bfloat16),
            pltpu.SemaphoreType.DMA((4,)),
            pltpu.SemaphoreType.DMA((4,)),
            pltpu.SemaphoreType.DMA((4,)),
            pltpu.SemaphoreType.DMA((4,)),
            pltpu.SemaphoreType.DMA((4,)),
            pltpu.SemaphoreType.DMA((4,)),
            pltpu.SemaphoreType.DMA((4,)),
            pltpu.SemaphoreType.DMA((4,)),
        ],
        compiler_params=pltpu.CompilerParams(collective_id=0),
    )(xb, wq, kh, vh, wo)
    return out.reshape(1, SQ, D_MODEL)


# device time: 80327 ns/iter; 1.0972x vs baseline; 1.0068x over previous
import jax
import jax.numpy as jnp
from jax import lax
from jax.experimental import pallas as pl
from jax.experimental.pallas import tpu as pltpu

N_DEV = 16
SQ = 1024
SKV = 1024
D_MODEL = 1024
HALF_D = D_MODEL // 2
HEADS_PER_SHARD = 8
DH = 128
WINDOW = 128
SCALE = 0.08838834764831843

MASKS_A = (1, 2, 4, 8)
MASKS_B = (4, 8, 2, 1)
CONTRIB_A = {1: 512, 2: 256, 4: 128, 8: 64}
CONTRIB_B = {4: 512, 8: 256, 2: 128, 1: 64}
HALVES = (512, 256, 128, 64)


def _body(x_ref, wq_ref, k_ref, v_ref, wo_ref, out_ref,
          q_ref, ctx_ref, acc_ref,
          accb_ref, rbufa_ref, rbufb_ref, gbufa_ref, gbufb_ref,
          rsa_send, rsa_recv, rsb_send, rsb_recv,
          dba_send, dba_recv, dbb_send, dbb_recv):
    my = lax.axis_index("i")

    q = lax.dot_general(
        x_ref[...], wq_ref[...], (((1,), (0,)), ((), ())),
        preferred_element_type=jnp.float32,
    )
    q_ref[...] = q.astype(jnp.bfloat16)

    RBLK = 256
    for h in range(HEADS_PER_SHARD):
        for r in range(SQ // RBLK):
            r0 = r * RBLK
            c0 = max(0, r0 - WINDOW)
            c1 = min(SKV, r0 + RBLK + WINDOW)
            w = c1 - c0
            qblk = q_ref[r0:r0 + RBLK, h * DH:(h + 1) * DH]
            scores = lax.dot_general(
                qblk, k_ref[h, c0:c1, :], (((1,), (1,)), ((), ())),
                preferred_element_type=jnp.float32,
            ) * SCALE
            rows = lax.broadcasted_iota(jnp.int32, (RBLK, w), 0) + r0
            cols = lax.broadcasted_iota(jnp.int32, (RBLK, w), 1) + c0
            scores = jnp.where(jnp.abs(rows - cols) <= WINDOW, scores, -1e9)
            m = jnp.max(scores, axis=1, keepdims=True)
            e = jnp.exp(scores - m)
            s = jnp.sum(e, axis=1, keepdims=True)
            wgt = (e / s).astype(jnp.bfloat16)
            ctx = lax.dot_general(
                wgt, v_ref[h, c0:c1, :], (((1,), (0,)), ((), ())),
                preferred_element_type=jnp.float32,
            )
            ctx_ref[r0:r0 + RBLK, h * DH:(h + 1) * DH] = ctx.astype(
                jnp.bfloat16
            )

    accv = lax.dot_general(
        ctx_ref[...], wo_ref[...], (((1,), (0,)), ((), ())),
        preferred_element_type=jnp.float32,
    )
    acc_ref[...] = accv
    accb_ref[...] = accv.astype(jnp.bfloat16)

    bsem = pltpu.get_barrier_semaphore()
    for mk in MASKS_A:
        pl.semaphore_signal(bsem, inc=1, device_id=(my ^ mk,),
                            device_id_type=pl.DeviceIdType.MESH)
    pl.semaphore_wait(bsem, 4)

    def _keep_give(start, half, mk):
        upper = (my & mk) != 0
        keep = pl.multiple_of(
            jnp.where(upper, start + half, start).astype(jnp.int32), 64
        )
        give = pl.multiple_of(
            jnp.where(upper, start, start + half).astype(jnp.int32), 64
        )
        return keep, give

    def _rs_issue(k, give, mk, cols, rbuf, send_sems, recv_sems):
        half = HALVES[k]
        step = pltpu.make_async_remote_copy(
            src_ref=accb_ref.at[pl.ds(give, half), cols],
            dst_ref=rbuf.at[k, 0:half, :],
            send_sem=send_sems.at[k], recv_sem=recv_sems.at[k],
            device_id=(my ^ mk,), device_id_type=pl.DeviceIdType.MESH,
        )
        step.start()
        return step

    COLS_A = slice(0, HALF_D)
    COLS_B = slice(HALF_D, D_MODEL)

    keeps_a = [None] * 4
    keeps_b = [None] * 4
    gives_a = [None] * 4
    gives_b = [None] * 4
    sa, sb = jnp.int32(0), jnp.int32(0)
    for k in range(4):
        keeps_a[k], gives_a[k] = _keep_give(sa, HALVES[k], MASKS_A[k])
        keeps_b[k], gives_b[k] = _keep_give(sb, HALVES[k], MASKS_B[k])
        sa, sb = keeps_a[k], keeps_b[k]
    start_a, start_b = sa, sb

    def _rs_add(k, keep, cols, rbuf):
        half = HALVES[k]
        new = (
            acc_ref[pl.ds(keep, half), cols]
            + rbuf[k, 0:half, :].astype(jnp.float32)
        )
        acc_ref[pl.ds(keep, half), cols] = new
        accb_ref[pl.ds(keep, half), cols] = new.astype(jnp.bfloat16)

    steps_a = [None] * 4
    steps_b = [None] * 4
    steps_a[0] = _rs_issue(0, gives_a[0], MASKS_A[0], COLS_A,
                           rbufa_ref, rsa_send, rsa_recv)
    steps_b[0] = _rs_issue(0, gives_b[0], MASKS_B[0], COLS_B,
                           rbufb_ref, rsb_send, rsb_recv)
    for k in range(4):
        steps_a[k].wait_recv()
        _rs_add(k, keeps_a[k], COLS_A, rbufa_ref)
        if k + 1 < 4:
            steps_a[k + 1] = _rs_issue(k + 1, gives_a[k + 1], MASKS_A[k + 1],
                                       COLS_A, rbufa_ref, rsa_send, rsa_recv)
        steps_b[k].wait_recv()
        _rs_add(k, keeps_b[k], COLS_B, rbufb_ref)
        if k + 1 < 4:
            steps_b[k + 1] = _rs_issue(k + 1, gives_b[k + 1], MASKS_B[k + 1],
                                       COLS_B, rbufb_ref, rsb_send, rsb_recv)
    for k in range(4):
        steps_a[k].wait_send()
        steps_b[k].wait_send()

    gbufa_ref[pl.ds(start_a, 64), :] = accb_ref[pl.ds(start_a, 64), COLS_A]
    gbufb_ref[pl.ds(start_b, 64), :] = accb_ref[pl.ds(start_b, 64), COLS_B]

    def _db_desc(k, start, ln, mk, gbuf, send_sems, recv_sems):
        return pltpu.make_async_remote_copy(
            src_ref=gbuf.at[pl.ds(start, ln), :],
            dst_ref=gbuf.at[pl.ds(start, ln), :],
            send_sem=send_sems.at[k], recv_sem=recv_sems.at[k],
            device_id=(my ^ mk,), device_id_type=pl.DeviceIdType.MESH,
        )

    cur_a, cur_b = start_a, start_b
    curs_a = [None] * 4
    curs_b = [None] * 4
    parts_a = [None] * 4
    parts_b = [None] * 4
    lens = (64, 128, 256, 512)
    for k in range(4):
        mka, mkb = MASKS_A[3 - k], MASKS_B[3 - k]
        curs_a[k], curs_b[k] = cur_a, cur_b
        parts_a[k] = pl.multiple_of(
            jnp.where((my & mka) != 0, cur_a - CONTRIB_A[mka],
                      cur_a + CONTRIB_A[mka]).astype(jnp.int32), 64)
        parts_b[k] = pl.multiple_of(
            jnp.where((my & mkb) != 0, cur_b - CONTRIB_B[mkb],
                      cur_b + CONTRIB_B[mkb]).astype(jnp.int32), 64)
        cur_a = pl.multiple_of(jnp.minimum(cur_a, parts_a[k]), 64)
        cur_b = pl.multiple_of(jnp.minimum(cur_b, parts_b[k]), 64)

    sends_a = [None] * 4
    sends_b = [None] * 4
    sends_a[0] = _db_desc(0, curs_a[0], lens[0], MASKS_A[3],
                          gbufa_ref, dba_send, dba_recv)
    sends_b[0] = _db_desc(0, curs_b[0], lens[0], MASKS_B[3],
                          gbufb_ref, dbb_send, dbb_recv)
    sends_a[0].start()
    sends_b[0].start()
    for k in range(4):
        mka, mkb = MASKS_A[3 - k], MASKS_B[3 - k]
        recv_a = _db_desc(k, parts_a[k], lens[k], mka,
                          gbufa_ref, dba_send, dba_recv)
        recv_a.wait_recv()
        if k + 1 < 4:
            sends_a[k + 1] = _db_desc(k + 1, curs_a[k + 1], lens[k + 1],
                                      MASKS_A[3 - k - 1],
                                      gbufa_ref, dba_send, dba_recv)
            sends_a[k + 1].start()
        recv_b = _db_desc(k, parts_b[k], lens[k], mkb,
                          gbufb_ref, dbb_send, dbb_recv)
        recv_b.wait_recv()
        if k + 1 < 4:
            sends_b[k + 1] = _db_desc(k + 1, curs_b[k + 1], lens[k + 1],
                                      MASKS_B[3 - k - 1],
                                      gbufb_ref, dbb_send, dbb_recv)
            sends_b[k + 1].start()
    for k in range(4):
        sends_a[k].wait_send()
        sends_b[k].wait_send()

    out_ref[:, COLS_A] = gbufa_ref[...]
    out_ref[:, COLS_B] = gbufb_ref[...]


def kernel(x, Wq, K_ext, V_ext, Wo):
    pos = lax.axis_index("i")
    xb = x[0].astype(jnp.bfloat16)
    wq = Wq.astype(jnp.bfloat16)
    wo = Wo.astype(jnp.bfloat16)
    kh = lax.dynamic_slice(
        K_ext, (0, 0, pos * HEADS_PER_SHARD, 0), (1, SKV, HEADS_PER_SHARD, DH)
    )[0]
    vh = lax.dynamic_slice(
        V_ext, (0, 0, pos * HEADS_PER_SHARD, 0), (1, SKV, HEADS_PER_SHARD, DH)
    )[0]
    kh = jnp.transpose(kh.astype(jnp.bfloat16), (1, 0, 2))
    vh = jnp.transpose(vh.astype(jnp.bfloat16), (1, 0, 2))

    out = pl.pallas_call(
        _body,
        out_shape=jax.ShapeDtypeStruct((SQ, D_MODEL), jnp.bfloat16),
        in_specs=[pl.BlockSpec(memory_space=pltpu.VMEM)] * 5,
        out_specs=pl.BlockSpec(memory_space=pltpu.VMEM),
        scratch_shapes=[
            pltpu.VMEM((SQ, D_MODEL), jnp.bfloat16),
            pltpu.VMEM((SQ, D_MODEL), jnp.bfloat16),
            pltpu.VMEM((SQ, D_MODEL), jnp.float32),
            pltpu.VMEM((SQ, D_MODEL), jnp.bfloat16),
            pltpu.VMEM((4, SQ // 2, HALF_D), jnp.bfloat16),
            pltpu.VMEM((4, SQ // 2, HALF_D), jnp.bfloat16),
            pltpu.VMEM((SQ, HALF_D), jnp.bfloat16),
            pltpu.VMEM((SQ, HALF_D), jnp.bfloat16),
            pltpu.SemaphoreType.DMA((4,)),
            pltpu.SemaphoreType.DMA((4,)),
            pltpu.SemaphoreType.DMA((4,)),
            pltpu.SemaphoreType.DMA((4,)),
            pltpu.SemaphoreType.DMA((4,)),
            pltpu.SemaphoreType.DMA((4,)),
            pltpu.SemaphoreType.DMA((4,)),
            pltpu.SemaphoreType.DMA((4,)),
        ],
        compiler_params=pltpu.CompilerParams(collective_id=0),
    )(xb, wq, kh, vh, wo)
    return out.reshape(1, SQ, D_MODEL)


# device time: 79661 ns/iter; 1.1064x vs baseline; 1.0084x over previous
import jax
import jax.numpy as jnp
from jax import lax
from jax.experimental import pallas as pl
from jax.experimental.pallas import tpu as pltpu

N_DEV = 16
SQ = 1024
SKV = 1024
D_MODEL = 1024
HALF_D = D_MODEL // 2
HALF_S = SQ // 2
HEADS_PER_SHARD = 8
DH = 128
WINDOW = 128
SCALE = 0.08838834764831843

MASKS_A = (1, 2, 4, 8)
MASKS_B = (4, 8, 2, 1)
CONTRIB_A = {1: 512, 2: 256, 4: 128, 8: 64}
CONTRIB_B = {4: 512, 8: 256, 2: 128, 1: 64}
HALVES = (512, 256, 128, 64)


def _body(x_ref, wq_ref, k_ref, v_ref, wo_ref, out_ref,
          q_ref, ctx_ref, acc_ref,
          accb_ref, rbufa_ref, rbufb_ref, gbufa_ref, gbufb_ref,
          rsa_send, rsa_recv, rsb_send, rsb_recv,
          dba_send, dba_recv, dbb_send, dbb_recv):
    my = lax.axis_index("i")

    bsem = pltpu.get_barrier_semaphore()
    for mk in MASKS_A:
        pl.semaphore_signal(bsem, inc=1, device_id=(my ^ mk,),
                            device_id_type=pl.DeviceIdType.MESH)
    pl.semaphore_wait(bsem, 4)

    def _keep_give(start, half, mk):
        upper = (my & mk) != 0
        keep = pl.multiple_of(
            jnp.where(upper, start + half, start).astype(jnp.int32), 64
        )
        give = pl.multiple_of(
            jnp.where(upper, start, start + half).astype(jnp.int32), 64
        )
        return keep, give

    keeps_a = [None] * 4
    keeps_b = [None] * 4
    gives_a = [None] * 4
    gives_b = [None] * 4
    sa, sb = jnp.int32(0), jnp.int32(0)
    for k in range(4):
        keeps_a[k], gives_a[k] = _keep_give(sa, HALVES[k], MASKS_A[k])
        keeps_b[k], gives_b[k] = _keep_give(sb, HALVES[k], MASKS_B[k])
        sa, sb = keeps_a[k], keeps_b[k]
    start_a, start_b = sa, sb

    COLS_A = slice(0, HALF_D)
    COLS_B = slice(HALF_D, D_MODEL)

    h1 = pl.multiple_of(gives_a[0], 512)
    h2 = pl.multiple_of(keeps_a[0], 512)

    def _compute_rows(base):
        qv = lax.dot_general(
            x_ref[pl.ds(base, HALF_S), :], wq_ref[...],
            (((1,), (0,)), ((), ())),
            preferred_element_type=jnp.float32,
        )
        q_ref[pl.ds(base, HALF_S), :] = qv.astype(jnp.bfloat16)
        RBLK = 256
        for h in range(HEADS_PER_SHARD):
            for r in range(2):
                r0 = pl.multiple_of(base + r * RBLK, RBLK)
                c0 = pl.multiple_of(
                    jnp.clip(r0 - WINDOW, 0, SKV - 512).astype(jnp.int32),
                    128,
                )
                qblk = q_ref[pl.ds(r0, RBLK), h * DH:(h + 1) * DH]
                scores = lax.dot_general(
                    qblk, k_ref[h, pl.ds(c0, 512), :],
                    (((1,), (1,)), ((), ())),
                    preferred_element_type=jnp.float32,
                ) * SCALE
                rows = lax.broadcasted_iota(jnp.int32, (RBLK, 512), 0) + r0
                cols = lax.broadcasted_iota(jnp.int32, (RBLK, 512), 1) + c0
                scores = jnp.where(
                    jnp.abs(rows - cols) <= WINDOW, scores, -1e9
                )
                m = jnp.max(scores, axis=1, keepdims=True)
                e = jnp.exp(scores - m)
                s = jnp.sum(e, axis=1, keepdims=True)
                wgt = (e / s).astype(jnp.bfloat16)
                ctx = lax.dot_general(
                    wgt, v_ref[h, pl.ds(c0, 512), :],
                    (((1,), (0,)), ((), ())),
                    preferred_element_type=jnp.float32,
                )
                ctx_ref[pl.ds(r0, RBLK), h * DH:(h + 1) * DH] = ctx.astype(
                    jnp.bfloat16
                )

    def _wo_quad(rbase, cols):
        val = lax.dot_general(
            ctx_ref[pl.ds(rbase, HALF_S), :], wo_ref[:, cols],
            (((1,), (0,)), ((), ())),
            preferred_element_type=jnp.float32,
        )
        acc_ref[pl.ds(rbase, HALF_S), cols] = val
        accb_ref[pl.ds(rbase, HALF_S), cols] = val.astype(jnp.bfloat16)

    def _rs_issue(k, give, mk, cols, rbuf, send_sems, recv_sems):
        half = HALVES[k]
        step = pltpu.make_async_remote_copy(
            src_ref=accb_ref.at[pl.ds(give, half), cols],
            dst_ref=rbuf.at[k, 0:half, :],
            send_sem=send_sems.at[k], recv_sem=recv_sems.at[k],
            device_id=(my ^ mk,), device_id_type=pl.DeviceIdType.MESH,
        )
        step.start()
        return step

    steps_a = [None] * 4
    steps_b = [None] * 4

    _compute_rows(h1)
    _wo_quad(h1, COLS_A)
    steps_a[0] = _rs_issue(0, gives_a[0], MASKS_A[0], COLS_A,
                           rbufa_ref, rsa_send, rsa_recv)
    _compute_rows(h2)
    _wo_quad(gives_b[0], COLS_B)
    steps_b[0] = _rs_issue(0, gives_b[0], MASKS_B[0], COLS_B,
                           rbufb_ref, rsb_send, rsb_recv)
    _wo_quad(h2, COLS_A)
    _wo_quad(keeps_b[0], COLS_B)

    def _rs_add(k, keep, cols, rbuf):
        half = HALVES[k]
        new = (
            acc_ref[pl.ds(keep, half), cols]
            + rbuf[k, 0:half, :].astype(jnp.float32)
        )
        acc_ref[pl.ds(keep, half), cols] = new
        accb_ref[pl.ds(keep, half), cols] = new.astype(jnp.bfloat16)

    for k in range(4):
        steps_a[k].wait_recv()
        _rs_add(k, keeps_a[k], COLS_A, rbufa_ref)
        if k + 1 < 4:
            steps_a[k + 1] = _rs_issue(k + 1, gives_a[k + 1], MASKS_A[k + 1],
                                       COLS_A, rbufa_ref, rsa_send, rsa_recv)
        steps_b[k].wait_recv()
        _rs_add(k, keeps_b[k], COLS_B, rbufb_ref)
        if k + 1 < 4:
            steps_b[k + 1] = _rs_issue(k + 1, gives_b[k + 1], MASKS_B[k + 1],
                                       COLS_B, rbufb_ref, rsb_send, rsb_recv)
    for k in range(4):
        steps_a[k].wait_send()
        steps_b[k].wait_send()

    gbufa_ref[pl.ds(start_a, 64), :] = accb_ref[pl.ds(start_a, 64), COLS_A]
    gbufb_ref[pl.ds(start_b, 64), :] = accb_ref[pl.ds(start_b, 64), COLS_B]

    def _db_desc(k, start, ln, mk, gbuf, send_sems, recv_sems):
        return pltpu.make_async_remote_copy(
            src_ref=gbuf.at[pl.ds(start, ln), :],
            dst_ref=gbuf.at[pl.ds(start, ln), :],
            send_sem=send_sems.at[k], recv_sem=recv_sems.at[k],
            device_id=(my ^ mk,), device_id_type=pl.DeviceIdType.MESH,
        )

    cur_a, cur_b = start_a, start_b
    curs_a = [None] * 4
    curs_b = [None] * 4
    parts_a = [None] * 4
    parts_b = [None] * 4
    lens = (64, 128, 256, 512)
    for k in range(4):
        mka, mkb = MASKS_A[3 - k], MASKS_B[3 - k]
        curs_a[k], curs_b[k] = cur_a, cur_b
        parts_a[k] = pl.multiple_of(
            jnp.where((my & mka) != 0, cur_a - CONTRIB_A[mka],
                      cur_a + CONTRIB_A[mka]).astype(jnp.int32), 64)
        parts_b[k] = pl.multiple_of(
            jnp.where((my & mkb) != 0, cur_b - CONTRIB_B[mkb],
                      cur_b + CONTRIB_B[mkb]).astype(jnp.int32), 64)
        cur_a = pl.multiple_of(jnp.minimum(cur_a, parts_a[k]), 64)
        cur_b = pl.multiple_of(jnp.minimum(cur_b, parts_b[k]), 64)

    sends_a = [None] * 4
    sends_b = [None] * 4
    sends_a[0] = _db_desc(0, curs_a[0], lens[0], MASKS_A[3],
                          gbufa_ref, dba_send, dba_recv)
    sends_b[0] = _db_desc(0, curs_b[0], lens[0], MASKS_B[3],
                          gbufb_ref, dbb_send, dbb_recv)
    sends_a[0].start()
    sends_b[0].start()
    for k in range(4):
        mka, mkb = MASKS_A[3 - k], MASKS_B[3 - k]
        recv_a = _db_desc(k, parts_a[k], lens[k], mka,
                          gbufa_ref, dba_send, dba_recv)
        recv_a.wait_recv()
        if k + 1 < 4:
            sends_a[k + 1] = _db_desc(k + 1, curs_a[k + 1], lens[k + 1],
                                      MASKS_A[3 - k - 1],
                                      gbufa_ref, dba_send, dba_recv)
            sends_a[k + 1].start()
        recv_b = _db_desc(k, parts_b[k], lens[k], mkb,
                          gbufb_ref, dbb_send, dbb_recv)
        recv_b.wait_recv()
        if k + 1 < 4:
            sends_b[k + 1] = _db_desc(k + 1, curs_b[k + 1], lens[k + 1],
                                      MASKS_B[3 - k - 1],
                                      gbufb_ref, dbb_send, dbb_recv)
            sends_b[k + 1].start()
    for k in range(4):
        sends_a[k].wait_send()
        sends_b[k].wait_send()

    out_ref[:, COLS_A] = gbufa_ref[...]
    out_ref[:, COLS_B] = gbufb_ref[...]


def kernel(x, Wq, K_ext, V_ext, Wo):
    pos = lax.axis_index("i")
    xb = x[0].astype(jnp.bfloat16)
    wq = Wq.astype(jnp.bfloat16)
    wo = Wo.astype(jnp.bfloat16)
    kh = lax.dynamic_slice(
        K_ext, (0, 0, pos * HEADS_PER_SHARD, 0), (1, SKV, HEADS_PER_SHARD, DH)
    )[0]
    vh = lax.dynamic_slice(
        V_ext, (0, 0, pos * HEADS_PER_SHARD, 0), (1, SKV, HEADS_PER_SHARD, DH)
    )[0]
    kh = jnp.transpose(kh.astype(jnp.bfloat16), (1, 0, 2))
    vh = jnp.transpose(vh.astype(jnp.bfloat16), (1, 0, 2))

    out = pl.pallas_call(
        _body,
        out_shape=jax.ShapeDtypeStruct((SQ, D_MODEL), jnp.bfloat16),
        in_specs=[pl.BlockSpec(memory_space=pltpu.VMEM)] * 5,
        out_specs=pl.BlockSpec(memory_space=pltpu.VMEM),
        scratch_shapes=[
            pltpu.VMEM((SQ, D_MODEL), jnp.bfloat16),
            pltpu.VMEM((SQ, D_MODEL), jnp.bfloat16),
            pltpu.VMEM((SQ, D_MODEL), jnp.float32),
            pltpu.VMEM((SQ, D_MODEL), jnp.bfloat16),
            pltpu.VMEM((4, SQ // 2, HALF_D), jnp.bfloat16),
            pltpu.VMEM((4, SQ // 2, HALF_D), jnp.bfloat16),
            pltpu.VMEM((SQ, HALF_D), jnp.bfloat16),
            pltpu.VMEM((SQ, HALF_D), jnp.bfloat16),
            pltpu.SemaphoreType.DMA((4,)),
            pltpu.SemaphoreType.DMA((4,)),
            pltpu.SemaphoreType.DMA((4,)),
            pltpu.SemaphoreType.DMA((4,)),
            pltpu.SemaphoreType.DMA((4,)),
            pltpu.SemaphoreType.DMA((4,)),
            pltpu.SemaphoreType.DMA((4,)),
            pltpu.SemaphoreType.DMA((4,)),
        ],
        compiler_params=pltpu.CompilerParams(collective_id=0),
    )(xb, wq, kh, vh, wo)
    return out.reshape(1, SQ, D_MODEL)


# device time: 75506 ns/iter; 1.1673x vs baseline; 1.0550x over previous
import jax
import jax.numpy as jnp
from jax import lax
from jax.experimental import pallas as pl
from jax.experimental.pallas import tpu as pltpu

N_DEV = 16
SQ = 1024
SKV = 1024
D_MODEL = 1024
HALF_D = D_MODEL // 2
HALF_S = SQ // 2
HEADS_PER_SHARD = 8
DH = 128
WINDOW = 128
SCALE = 0.08838834764831843

MASKS_A = (1, 2, 4, 8)
MASKS_B = (4, 8, 2, 1)
CONTRIB_A = {1: 512, 2: 256, 4: 128, 8: 64}
CONTRIB_B = {4: 512, 8: 256, 2: 128, 1: 64}
HALVES = (512, 256, 128, 64)


def _body(x_ref, wq_ref, khbm_ref, vhbm_ref, wo_ref, out_ref,
          q_ref, ctx_ref, acc_ref,
          ktb_ref, vtb_ref, ktf_ref, vtf_ref, kv_sems,
          accb_ref, rbufa_ref, rbufb_ref, gbufa_ref, gbufb_ref,
          rsa_send, rsa_recv, rsb_send, rsb_recv,
          dba_send, dba_recv, dbb_send, dbb_recv):
    my = lax.axis_index("i")

    g0 = my * HEADS_PER_SHARD
    kv_loads = []
    for h in range(HEADS_PER_SHARD):
        ck = pltpu.make_async_copy(
            khbm_ref.at[0, :, g0 + h, :], ktf_ref.at[h], kv_sems.at[h]
        )
        ck.start()
        cv = pltpu.make_async_copy(
            vhbm_ref.at[0, :, g0 + h, :], vtf_ref.at[h],
            kv_sems.at[HEADS_PER_SHARD + h],
        )
        cv.start()
        kv_loads.append((ck, cv))

    bsem = pltpu.get_barrier_semaphore()
    for mk in MASKS_A:
        pl.semaphore_signal(bsem, inc=1, device_id=(my ^ mk,),
                            device_id_type=pl.DeviceIdType.MESH)
    pl.semaphore_wait(bsem, 4)

    def _keep_give(start, half, mk):
        upper = (my & mk) != 0
        keep = pl.multiple_of(
            jnp.where(upper, start + half, start).astype(jnp.int32), 64
        )
        give = pl.multiple_of(
            jnp.where(upper, start, start + half).astype(jnp.int32), 64
        )
        return keep, give

    keeps_a = [None] * 4
    keeps_b = [None] * 4
    gives_a = [None] * 4
    gives_b = [None] * 4
    sa, sb = jnp.int32(0), jnp.int32(0)
    for k in range(4):
        keeps_a[k], gives_a[k] = _keep_give(sa, HALVES[k], MASKS_A[k])
        keeps_b[k], gives_b[k] = _keep_give(sb, HALVES[k], MASKS_B[k])
        sa, sb = keeps_a[k], keeps_b[k]
    start_a, start_b = sa, sb

    COLS_A = slice(0, HALF_D)
    COLS_B = slice(HALF_D, D_MODEL)

    h1 = pl.multiple_of(gives_a[0], 512)
    h2 = pl.multiple_of(keeps_a[0], 512)

    def _compute_rows(base, first):
        qv = lax.dot_general(
            x_ref[pl.ds(base, HALF_S), :], wq_ref[...],
            (((1,), (0,)), ((), ())),
            preferred_element_type=jnp.float32,
        )
        q_ref[pl.ds(base, HALF_S), :] = qv.astype(jnp.bfloat16)
        RBLK = 256
        for h in range(HEADS_PER_SHARD):
            if first:
                ck, cv = kv_loads[h]
                ck.wait()
                cv.wait()
                ktb_ref[h, :, :] = ktf_ref[h, :, :].astype(jnp.bfloat16)
                vtb_ref[h, :, :] = vtf_ref[h, :, :].astype(jnp.bfloat16)
            for r in range(2):
                r0 = pl.multiple_of(base + r * RBLK, RBLK)
                c0 = pl.multiple_of(
                    jnp.clip(r0 - WINDOW, 0, SKV - 512).astype(jnp.int32),
                    128,
                )
                qblk = q_ref[pl.ds(r0, RBLK), h * DH:(h + 1) * DH]
                scores = lax.dot_general(
                    qblk, ktb_ref[h, pl.ds(c0, 512), :],
                    (((1,), (1,)), ((), ())),
                    preferred_element_type=jnp.float32,
                ) * SCALE
                rows = lax.broadcasted_iota(jnp.int32, (RBLK, 512), 0) + r0
                cols = lax.broadcasted_iota(jnp.int32, (RBLK, 512), 1) + c0
                scores = jnp.where(
                    jnp.abs(rows - cols) <= WINDOW, scores, -1e9
                )
                m = jnp.max(scores, axis=1, keepdims=True)
                e = jnp.exp(scores - m)
                s = jnp.sum(e, axis=1, keepdims=True)
                wgt = (e / s).astype(jnp.bfloat16)
                ctx = lax.dot_general(
                    wgt, vtb_ref[h, pl.ds(c0, 512), :],
                    (((1,), (0,)), ((), ())),
                    preferred_element_type=jnp.float32,
                )
                ctx_ref[pl.ds(r0, RBLK), h * DH:(h + 1) * DH] = ctx.astype(
                    jnp.bfloat16
                )

    def _wo_quad(rbase, cols):
        val = lax.dot_general(
            ctx_ref[pl.ds(rbase, HALF_S), :], wo_ref[:, cols],
            (((1,), (0,)), ((), ())),
            preferred_element_type=jnp.float32,
        )
        acc_ref[pl.ds(rbase, HALF_S), cols] = val
        accb_ref[pl.ds(rbase, HALF_S), cols] = val.astype(jnp.bfloat16)

    def _rs_issue(k, give, mk, cols, rbuf, send_sems, recv_sems):
        half = HALVES[k]
        step = pltpu.make_async_remote_copy(
            src_ref=accb_ref.at[pl.ds(give, half), cols],
            dst_ref=rbuf.at[k, 0:half, :],
            send_sem=send_sems.at[k], recv_sem=recv_sems.at[k],
            device_id=(my ^ mk,), device_id_type=pl.DeviceIdType.MESH,
        )
        step.start()
        return step

    steps_a = [None] * 4
    steps_b = [None] * 4

    _compute_rows(h1, first=True)
    _wo_quad(h1, COLS_A)
    steps_a[0] = _rs_issue(0, gives_a[0], MASKS_A[0], COLS_A,
                           rbufa_ref, rsa_send, rsa_recv)
    _compute_rows(h2, first=False)
    _wo_quad(gives_b[0], COLS_B)
    steps_b[0] = _rs_issue(0, gives_b[0], MASKS_B[0], COLS_B,
                           rbufb_ref, rsb_send, rsb_recv)
    _wo_quad(h2, COLS_A)
    _wo_quad(keeps_b[0], COLS_B)

    def _rs_add(k, keep, cols, rbuf):
        half = HALVES[k]
        new = (
            acc_ref[pl.ds(keep, half), cols]
            + rbuf[k, 0:half, :].astype(jnp.float32)
        )
        acc_ref[pl.ds(keep, half), cols] = new
        accb_ref[pl.ds(keep, half), cols] = new.astype(jnp.bfloat16)

    for k in range(4):
        steps_a[k].wait_recv()
        _rs_add(k, keeps_a[k], COLS_A, rbufa_ref)
        if k + 1 < 4:
            steps_a[k + 1] = _rs_issue(k + 1, gives_a[k + 1], MASKS_A[k + 1],
                                       COLS_A, rbufa_ref, rsa_send, rsa_recv)
        steps_b[k].wait_recv()
        _rs_add(k, keeps_b[k], COLS_B, rbufb_ref)
        if k + 1 < 4:
            steps_b[k + 1] = _rs_issue(k + 1, gives_b[k + 1], MASKS_B[k + 1],
                                       COLS_B, rbufb_ref, rsb_send, rsb_recv)
    for k in range(4):
        steps_a[k].wait_send()
        steps_b[k].wait_send()

    gbufa_ref[pl.ds(start_a, 64), :] = accb_ref[pl.ds(start_a, 64), COLS_A]
    gbufb_ref[pl.ds(start_b, 64), :] = accb_ref[pl.ds(start_b, 64), COLS_B]

    def _db_desc(k, start, ln, mk, gbuf, send_sems, recv_sems):
        return pltpu.make_async_remote_copy(
            src_ref=gbuf.at[pl.ds(start, ln), :],
            dst_ref=gbuf.at[pl.ds(start, ln), :],
            send_sem=send_sems.at[k], recv_sem=recv_sems.at[k],
            device_id=(my ^ mk,), device_id_type=pl.DeviceIdType.MESH,
        )

    cur_a, cur_b = start_a, start_b
    curs_a = [None] * 4
    curs_b = [None] * 4
    parts_a = [None] * 4
    parts_b = [None] * 4
    lens = (64, 128, 256, 512)
    for k in range(4):
        mka, mkb = MASKS_A[3 - k], MASKS_B[3 - k]
        curs_a[k], curs_b[k] = cur_a, cur_b
        parts_a[k] = pl.multiple_of(
            jnp.where((my & mka) != 0, cur_a - CONTRIB_A[mka],
                      cur_a + CONTRIB_A[mka]).astype(jnp.int32), 64)
        parts_b[k] = pl.multiple_of(
            jnp.where((my & mkb) != 0, cur_b - CONTRIB_B[mkb],
                      cur_b + CONTRIB_B[mkb]).astype(jnp.int32), 64)
        cur_a = pl.multiple_of(jnp.minimum(cur_a, parts_a[k]), 64)
        cur_b = pl.multiple_of(jnp.minimum(cur_b, parts_b[k]), 64)

    sends_a = [None] * 4
    sends_b = [None] * 4
    sends_a[0] = _db_desc(0, curs_a[0], lens[0], MASKS_A[3],
                          gbufa_ref, dba_send, dba_recv)
    sends_b[0] = _db_desc(0, curs_b[0], lens[0], MASKS_B[3],
                          gbufb_ref, dbb_send, dbb_recv)
    sends_a[0].start()
    sends_b[0].start()
    for k in range(4):
        mka, mkb = MASKS_A[3 - k], MASKS_B[3 - k]
        recv_a = _db_desc(k, parts_a[k], lens[k], mka,
                          gbufa_ref, dba_send, dba_recv)
        recv_a.wait_recv()
        if k + 1 < 4:
            sends_a[k + 1] = _db_desc(k + 1, curs_a[k + 1], lens[k + 1],
                                      MASKS_A[3 - k - 1],
                                      gbufa_ref, dba_send, dba_recv)
            sends_a[k + 1].start()
        recv_b = _db_desc(k, parts_b[k], lens[k], mkb,
                          gbufb_ref, dbb_send, dbb_recv)
        recv_b.wait_recv()
        if k + 1 < 4:
            sends_b[k + 1] = _db_desc(k + 1, curs_b[k + 1], lens[k + 1],
                                      MASKS_B[3 - k - 1],
                                      gbufb_ref, dbb_send, dbb_recv)
            sends_b[k + 1].start()
    for k in range(4):
        sends_a[k].wait_send()
        sends_b[k].wait_send()

    out_ref[:, COLS_A] = gbufa_ref[...]
    out_ref[:, COLS_B] = gbufb_ref[...]


def kernel(x, Wq, K_ext, V_ext, Wo):
    pos = lax.axis_index("i")
    xb = x[0].astype(jnp.bfloat16)
    wq = Wq.astype(jnp.bfloat16)
    wo = Wo.astype(jnp.bfloat16)
    out = pl.pallas_call(
        _body,
        out_shape=jax.ShapeDtypeStruct((SQ, D_MODEL), jnp.bfloat16),
        in_specs=[
            pl.BlockSpec(memory_space=pltpu.VMEM),
            pl.BlockSpec(memory_space=pltpu.VMEM),
            pl.BlockSpec(memory_space=pltpu.MemorySpace.HBM),
            pl.BlockSpec(memory_space=pltpu.MemorySpace.HBM),
            pl.BlockSpec(memory_space=pltpu.VMEM),
        ],
        out_specs=pl.BlockSpec(memory_space=pltpu.VMEM),
        scratch_shapes=[
            pltpu.VMEM((SQ, D_MODEL), jnp.bfloat16),
            pltpu.VMEM((SQ, D_MODEL), jnp.bfloat16),
            pltpu.VMEM((SQ, D_MODEL), jnp.float32),
            pltpu.VMEM((HEADS_PER_SHARD, SKV, DH), jnp.bfloat16),
            pltpu.VMEM((HEADS_PER_SHARD, SKV, DH), jnp.bfloat16),
            pltpu.VMEM((HEADS_PER_SHARD, SKV, DH), jnp.float32),
            pltpu.VMEM((HEADS_PER_SHARD, SKV, DH), jnp.float32),
            pltpu.SemaphoreType.DMA((2 * HEADS_PER_SHARD,)),
            pltpu.VMEM((SQ, D_MODEL), jnp.bfloat16),
            pltpu.VMEM((4, SQ // 2, HALF_D), jnp.bfloat16),
            pltpu.VMEM((4, SQ // 2, HALF_D), jnp.bfloat16),
            pltpu.VMEM((SQ, HALF_D), jnp.bfloat16),
            pltpu.VMEM((SQ, HALF_D), jnp.bfloat16),
            pltpu.SemaphoreType.DMA((4,)),
            pltpu.SemaphoreType.DMA((4,)),
            pltpu.SemaphoreType.DMA((4,)),
            pltpu.SemaphoreType.DMA((4,)),
            pltpu.SemaphoreType.DMA((4,)),
            pltpu.SemaphoreType.DMA((4,)),
            pltpu.SemaphoreType.DMA((4,)),
            pltpu.SemaphoreType.DMA((4,)),
        ],
        compiler_params=pltpu.CompilerParams(collective_id=0),
    )(xb, wq, K_ext, V_ext, wo)
    return out.reshape(1, SQ, D_MODEL)
